# Initial kernel scaffold; baseline (speedup 1.0000x reference)
import jax
import jax.numpy as jnp
from jax.experimental import pallas as pl


def kernel(x, pos_embedding):
    raise NotImplementedError("write your pallas kernel here")


# speedup vs baseline: 2.8483x; 2.8483x over previous
import jax
import jax.numpy as jnp
from jax.experimental import pallas as pl


def _add_kernel(x_ref, pe_ref, o_ref):
    o_ref[...] = x_ref[...] + pe_ref[...]


def kernel(x, pos_embedding):
    B, T, D = x.shape
    pe = pos_embedding[:T]
    BT = 512
    grid = (T // BT, B)
    return pl.pallas_call(
        _add_kernel,
        grid=grid,
        in_specs=[
            pl.BlockSpec((1, BT, D), lambda t, b: (b, t, 0)),
            pl.BlockSpec((BT, D), lambda t, b: (t, 0)),
        ],
        out_specs=pl.BlockSpec((1, BT, D), lambda t, b: (b, t, 0)),
        out_shape=jax.ShapeDtypeStruct((B, T, D), x.dtype),
    )(x, pe)


# speedup vs baseline: 3.1703x; 1.1130x over previous
import jax
import jax.numpy as jnp
from jax.experimental import pallas as pl


def _add_kernel(x_ref, pe_ref, o_ref):
    o_ref[...] = x_ref[...] + pe_ref[...]


def kernel(x, pos_embedding):
    B, T, D = x.shape
    pe = pos_embedding[:T]
    BT = 1024
    grid = (T // BT, B)
    return pl.pallas_call(
        _add_kernel,
        grid=grid,
        in_specs=[
            pl.BlockSpec((1, BT, D), lambda t, b: (b, t, 0)),
            pl.BlockSpec((BT, D), lambda t, b: (t, 0)),
        ],
        out_specs=pl.BlockSpec((1, BT, D), lambda t, b: (b, t, 0)),
        out_shape=jax.ShapeDtypeStruct((B, T, D), x.dtype),
    )(x, pe)


# speedup vs baseline: 3.3139x; 1.0453x over previous
import jax
import jax.numpy as jnp
from jax.experimental import pallas as pl


def _add_kernel(x_ref, pe_ref, o_ref):
    o_ref[...] = x_ref[...] + pe_ref[...]


def kernel(x, pos_embedding):
    B, T, D = x.shape
    pe = pos_embedding[:T]
    BT = 2048
    grid = (T // BT, B)
    return pl.pallas_call(
        _add_kernel,
        grid=grid,
        in_specs=[
            pl.BlockSpec((1, BT, D), lambda t, b: (b, t, 0)),
            pl.BlockSpec((BT, D), lambda t, b: (t, 0)),
        ],
        out_specs=pl.BlockSpec((1, BT, D), lambda t, b: (b, t, 0)),
        out_shape=jax.ShapeDtypeStruct((B, T, D), x.dtype),
    )(x, pe)
